# baseline (speedup 1.0000x reference)
import functools

import jax
import jax.numpy as jnp
from jax.experimental import pallas as pl
from jax.experimental.pallas import tpu as pltpu

T = 128
D = 1024
F = 512
E = 64
K = 8
NG = 8
TKG = 4
SCALE = 2.5

EPB = 2
NI = E // EPB
NBUF = 3


def _dot_nt(a, b):
    return jax.lax.dot_general(
        a, b, (((1,), (1,)), ((), ())), preferred_element_type=jnp.float32)


def _first_argmax(x, iota, width):
    m = jnp.max(x, axis=1, keepdims=True)
    idx = jnp.min(jnp.where(x >= m, iota, width), axis=1, keepdims=True)
    return iota == idx, m


def _router_combine(x, gw, bias):
    logits = _dot_nt(x, gw)
    scores = jax.nn.sigmoid(logits)
    sc = scores + bias

    gsz = E // NG
    iota_g = jax.lax.broadcasted_iota(jnp.int32, (T, gsz), 1)
    group_cols = []
    for g in range(NG):
        s = sc[:, g * gsz:(g + 1) * gsz]
        one1, m1 = _first_argmax(s, iota_g, gsz)
        s2 = jnp.where(one1, -jnp.inf, s)
        m2 = jnp.max(s2, axis=1, keepdims=True)
        group_cols.append(m1 + m2)
    gs = jnp.concatenate(group_cols, axis=1)

    iota_ng = jax.lax.broadcasted_iota(jnp.int32, (T, NG), 1)
    gmask = jnp.zeros((T, NG), dtype=jnp.float32)
    for _ in range(TKG):
        one, _m = _first_argmax(gs, iota_ng, NG)
        gmask = gmask + one.astype(jnp.float32)
        gs = jnp.where(one, -jnp.inf, gs)

    smask = jnp.concatenate(
        [jnp.broadcast_to(gmask[:, g:g + 1], (T, gsz)) for g in range(NG)],
        axis=1)
    scm = jnp.where(smask > 0.0, sc, -jnp.inf)

    iota_e = jax.lax.broadcasted_iota(jnp.int32, (T, E), 1)
    combine = jnp.zeros((T, E), dtype=jnp.float32)
    wsum = jnp.zeros((T, 1), dtype=jnp.float32)
    for _ in range(K):
        one, _m = _first_argmax(scm, iota_e, E)
        w = jnp.sum(jnp.where(one, scores, 0.0), axis=1, keepdims=True)
        combine = combine + jnp.where(one, w, 0.0)
        wsum = wsum + w
        scm = jnp.where(one, -jnp.inf, scm)
    return combine * (SCALE / (wsum + 1e-20))


def _moe_kernel(x_ref, gw_ref, bias_ref, w1_hbm, w2_hbm, w3_hbm,
                sw1_ref, sw2_ref, sw3_ref, out_ref,
                w1b, w2b, w3b, sems):
    def _copies(i, slot):
        sl = pl.ds(i * EPB, EPB)
        return (
            pltpu.make_async_copy(w1_hbm.at[sl], w1b.at[slot], sems.at[slot, 0]),
            pltpu.make_async_copy(w3_hbm.at[sl], w3b.at[slot], sems.at[slot, 1]),
            pltpu.make_async_copy(w2_hbm.at[sl], w2b.at[slot], sems.at[slot, 2]),
        )

    def _start(i, slot):
        for c in _copies(i, slot):
            c.start()

    for s in range(NBUF):
        _start(s, s)

    x = x_ref[...]
    combine = _router_combine(x, gw_ref[...], bias_ref[...])
    sh = jax.nn.silu(_dot_nt(x, sw1_ref[...])) * _dot_nt(x, sw3_ref[...])
    out_ref[...] = _dot_nt(sh, sw2_ref[...])

    iota_e = jax.lax.broadcasted_iota(jnp.int32, (T, E), 1)

    def _body(i, _):
        slot = jax.lax.rem(i, NBUF)
        c1, c3, c2 = _copies(i, slot)
        acc = jnp.zeros((T, D), dtype=jnp.float32)
        c1.wait()
        h1 = [_dot_nt(x, w1b[slot, j]) for j in range(EPB)]
        c3.wait()
        h = [jax.nn.silu(h1[j]) * _dot_nt(x, w3b[slot, j]) for j in range(EPB)]
        c2.wait()
        for j in range(EPB):
            y = _dot_nt(h[j], w2b[slot, j])
            col = jnp.sum(
                jnp.where(iota_e == i * EPB + j, combine, 0.0),
                axis=1, keepdims=True)
            acc = acc + y * col
        out_ref[...] += acc

        @pl.when(i + NBUF < NI)
        def _rearm():
            _start(i + NBUF, slot)
        return 0

    jax.lax.fori_loop(0, NI, _body, 0)


@jax.jit
def kernel(hidden_states, gate_weight, e_score_correction_bias,
           w1, w2, w3, sw1, sw2, sw3):
    orig_shape = hidden_states.shape
    x = hidden_states.reshape(T, D)
    bias = e_score_correction_bias.reshape(1, E)

    vmem = functools.partial(pl.BlockSpec, memory_space=pltpu.MemorySpace.VMEM)
    hbm = functools.partial(pl.BlockSpec, memory_space=pltpu.MemorySpace.HBM)

    out = pl.pallas_call(
        _moe_kernel,
        in_specs=[vmem(), vmem(), vmem(), hbm(), hbm(), hbm(),
                  vmem(), vmem(), vmem()],
        out_specs=vmem(),
        out_shape=jax.ShapeDtypeStruct((T, D), jnp.float32),
        scratch_shapes=[
            pltpu.VMEM((NBUF, EPB, F, D), jnp.float32),
            pltpu.VMEM((NBUF, EPB, D, F), jnp.float32),
            pltpu.VMEM((NBUF, EPB, F, D), jnp.float32),
            pltpu.SemaphoreType.DMA((NBUF, 3)),
        ],
        compiler_params=pltpu.CompilerParams(
            vmem_limit_bytes=100 * 1024 * 1024),
    )(x, gate_weight, bias, w1, w2, w3, sw1, sw2, sw3)
    return out.reshape(orig_shape)
